# Initial kernel scaffold; baseline (speedup 1.0000x reference)
import functools

import jax
import jax.numpy as jnp
from jax import lax
from jax.experimental import pallas as pl
from jax.experimental.pallas import tpu as pltpu
from jax.experimental.pallas import tpu_sc as plsc

N = 10000
E = 320000
D1 = 128
D2 = 40

NC = 2
NS = 16
L = 16
NW = NC * NS

EPW = E // NW
CHUNK = 80
NCHUNK = EPW // CHUNK
RPT = N // NS
NPAD = 10240
DPT = NPAD // NS

_MESH = plsc.VectorSubcoreMesh(
    core_axis_name="c", subcore_axis_name="s", num_cores=NC, num_subcores=NS)


@functools.partial(
    pl.kernel,
    out_type=jax.ShapeDtypeStruct((NC, NPAD), jnp.float32),
    mesh=_MESH,
    scratch_types=[
        pltpu.VMEM((CHUNK,), jnp.int32),
        pltpu.VMEM((CHUNK,), jnp.float32),
        pltpu.VMEM((DPT,), jnp.float32),
        pltpu.VMEM_SHARED((NPAD,), jnp.float32),
    ],
)
def _sc_degree(edge_hbm, out_hbm, col_v, ones_v, zfill_v, acc):
    cid = lax.axis_index("c")
    sid = lax.axis_index("s")
    wid = cid * NS + sid

    for j in range(CHUNK // L):
        ones_v[pl.ds(j * L, L)] = jnp.ones((L,), jnp.float32)
    for j in range(DPT // L):
        zfill_v[pl.ds(j * L, L)] = jnp.zeros((L,), jnp.float32)
    pltpu.sync_copy(zfill_v, acc.at[pl.ds(sid * DPT, DPT)])
    plsc.subcore_barrier()

    def chunk(g, carry):
        base = wid * EPW + g * CHUNK
        pltpu.sync_copy(edge_hbm.at[1, pl.ds(base, CHUNK)], col_v)
        pltpu.sync_copy(ones_v, acc.at[col_v], add=True)
        return carry

    lax.fori_loop(0, NCHUNK, chunk, 0)
    plsc.subcore_barrier()
    pltpu.sync_copy(acc.at[pl.ds(sid * DPT, DPT)],
                    out_hbm.at[cid, pl.ds(sid * DPT, DPT)])


def _make_sc_prop(d):
    @functools.partial(
        pl.kernel,
        out_type=jax.ShapeDtypeStruct((NC, N, d), jnp.float32),
        mesh=_MESH,
        scratch_types=[
            pltpu.VMEM((CHUNK,), jnp.int32),
            pltpu.VMEM((CHUNK,), jnp.int32),
            pltpu.VMEM((CHUNK, d), jnp.float32),
            pltpu.VMEM_SHARED((N, d), jnp.float32),
            pltpu.SemaphoreType.DMA,
        ],
    )
    def _sc_prop(edge_hbm, y_hbm, zeros_hbm, out_hbm,
                 row_v, col_v, rows_v, acc, sem):
        cid = lax.axis_index("c")
        sid = lax.axis_index("s")
        wid = cid * NS + sid

        pltpu.sync_copy(zeros_hbm, acc.at[pl.ds(sid * RPT, RPT)])
        plsc.subcore_barrier()

        def chunk(g, carry):
            base = wid * EPW + g * CHUNK
            pltpu.sync_copy(edge_hbm.at[0, pl.ds(base, CHUNK)], row_v)
            pltpu.sync_copy(edge_hbm.at[1, pl.ds(base, CHUNK)], col_v)
            pltpu.async_copy(y_hbm.at[row_v], rows_v, sem).wait()
            pltpu.sync_copy(rows_v, acc.at[col_v], add=True)
            return carry

        lax.fori_loop(0, NCHUNK, chunk, 0)
        plsc.subcore_barrier()
        pltpu.sync_copy(acc.at[pl.ds(sid * RPT, RPT)],
                        out_hbm.at[cid, pl.ds(sid * RPT, RPT)])

    return _sc_prop


_sc_prop1 = _make_sc_prop(D1)
_sc_prop2 = _make_sc_prop(D2)


def _tc_a_body(x_ref, w1_ref, degp_ref, y_ref, dinv_ref):
    deg = degp_ref[0] + degp_ref[1] + 1.0
    dinv = lax.rsqrt(deg)
    dinv_ref[...] = dinv
    xw = jnp.dot(x_ref[...], w1_ref[...], preferred_element_type=jnp.float32)
    y_ref[...] = xw * dinv[:N]


def _tc_b_body(accp_ref, y_ref, dinv_ref, b1_ref, gamma_ref, beta_ref,
               w2_ref, z_ref):
    dinv = dinv_ref[:N]
    s = dinv * (accp_ref[0] + accp_ref[1] + y_ref[...]) + b1_ref[...]
    mean = jnp.mean(s, axis=0, keepdims=True)
    var = jnp.mean((s - mean) * (s - mean), axis=0, keepdims=True)
    h = (s - mean) * (gamma_ref[...] / jnp.sqrt(var + 1e-5)) + beta_ref[...]
    h = jnp.maximum(h, 0.0)
    z_ref[...] = jnp.dot(h, w2_ref[...], preferred_element_type=jnp.float32) * dinv


def _tc_c_body(accp_ref, z_ref, dinv_ref, b2_ref, out_ref):
    out_ref[...] = (dinv_ref[:N] * (accp_ref[0] + accp_ref[1] + z_ref[...])
                    + b2_ref[...])


def kernel(x, edge_index, W1, b1, gamma, beta, W2, b2):
    degp = _sc_degree(edge_index)
    degp = degp.reshape(NC, NPAD, 1)

    y, dinv = pl.pallas_call(
        _tc_a_body,
        out_shape=(jax.ShapeDtypeStruct((N, D1), jnp.float32),
                   jax.ShapeDtypeStruct((NPAD, 1), jnp.float32)),
    )(x, W1, degp)

    zeros1 = jnp.zeros((RPT, D1), jnp.float32)
    accp1 = _sc_prop1(edge_index, y, zeros1)

    z = pl.pallas_call(
        _tc_b_body,
        out_shape=jax.ShapeDtypeStruct((N, D2), jnp.float32),
    )(accp1, y, dinv, b1.reshape(1, D1), gamma.reshape(1, D1),
      beta.reshape(1, D1), W2)

    zeros2 = jnp.zeros((RPT, D2), jnp.float32)
    accp2 = _sc_prop2(edge_index, z, zeros2)

    out = pl.pallas_call(
        _tc_c_body,
        out_shape=jax.ShapeDtypeStruct((N, D2), jnp.float32),
    )(accp2, z, dinv, b2.reshape(1, D2))
    return out


# speedup vs baseline: 13.1171x; 13.1171x over previous
import functools

import jax
import jax.numpy as jnp
from jax import lax
from jax.experimental import pallas as pl
from jax.experimental.pallas import tpu as pltpu
from jax.experimental.pallas import tpu_sc as plsc

N = 10000
E = 320000
D1 = 128
D2 = 40

NC = 2
NS = 16
L = 16
NW = NC * NS

EPW = E // NW
CHUNK = 80
NCHUNK = EPW // CHUNK
NPAD = 10240
RPT = NPAD // NS
DPT = NPAD // NS

_MESH = plsc.VectorSubcoreMesh(
    core_axis_name="c", subcore_axis_name="s", num_cores=NC, num_subcores=NS)


@functools.partial(
    pl.kernel,
    out_type=jax.ShapeDtypeStruct((NC, NPAD), jnp.float32),
    mesh=_MESH,
    scratch_types=[
        pltpu.VMEM((CHUNK,), jnp.int32),
        pltpu.VMEM((CHUNK,), jnp.float32),
        pltpu.VMEM((DPT,), jnp.float32),
        pltpu.VMEM_SHARED((NPAD,), jnp.float32),
    ],
)
def _sc_degree(edge_hbm, out_hbm, col_v, ones_v, zfill_v, acc):
    cid = lax.axis_index("c")
    sid = lax.axis_index("s")
    wid = cid * NS + sid

    for j in range(CHUNK // L):
        ones_v[pl.ds(j * L, L)] = jnp.ones((L,), jnp.float32)
    for j in range(DPT // L):
        zfill_v[pl.ds(j * L, L)] = jnp.zeros((L,), jnp.float32)
    pltpu.sync_copy(zfill_v, acc.at[pl.ds(sid * DPT, DPT)])
    plsc.subcore_barrier()

    def chunk(g, carry):
        base = wid * EPW + g * CHUNK
        pltpu.sync_copy(edge_hbm.at[pl.ds(E + base, CHUNK)], col_v)
        pltpu.sync_copy(ones_v, acc.at[col_v], add=True)
        return carry

    lax.fori_loop(0, NCHUNK, chunk, 0)
    plsc.subcore_barrier()
    pltpu.sync_copy(acc.at[pl.ds(sid * DPT, DPT)],
                    out_hbm.at[cid, pl.ds(sid * DPT, DPT)])


def _make_sc_prop(d):
    @functools.partial(
        pl.kernel,
        out_type=jax.ShapeDtypeStruct((NC, NPAD, d), jnp.float32),
        mesh=_MESH,
        scratch_types=[
            pltpu.VMEM((CHUNK,), jnp.int32),
            pltpu.VMEM((CHUNK,), jnp.int32),
            pltpu.VMEM((CHUNK, d), jnp.float32),
            pltpu.VMEM_SHARED((NPAD, d), jnp.float32),
            pltpu.SemaphoreType.DMA,
        ],
    )
    def _sc_prop(edge_hbm, y_hbm, zeros_hbm, out_hbm,
                 row_v, col_v, rows_v, acc, sem):
        cid = lax.axis_index("c")
        sid = lax.axis_index("s")
        wid = cid * NS + sid

        pltpu.sync_copy(zeros_hbm, acc.at[pl.ds(sid * RPT, RPT)])
        plsc.subcore_barrier()

        def chunk(g, carry):
            base = wid * EPW + g * CHUNK
            pltpu.sync_copy(edge_hbm.at[pl.ds(base, CHUNK)], row_v)
            pltpu.sync_copy(edge_hbm.at[pl.ds(E + base, CHUNK)], col_v)
            pltpu.async_copy(y_hbm.at[row_v], rows_v, sem).wait()
            pltpu.sync_copy(rows_v, acc.at[col_v], add=True)
            return carry

        lax.fori_loop(0, NCHUNK, chunk, 0)
        plsc.subcore_barrier()
        pltpu.sync_copy(acc.at[pl.ds(sid * RPT, RPT)],
                        out_hbm.at[cid, pl.ds(sid * RPT, RPT)])

    return _sc_prop


_sc_prop1 = _make_sc_prop(D1)
_sc_prop2 = _sc_prop1


def _tc_a_body(x_ref, w1_ref, degp_ref, y_ref, dinv_ref):
    deg = degp_ref[0] + degp_ref[1] + 1.0
    dinv = lax.rsqrt(deg)
    dinv_ref[...] = dinv
    xw = jnp.dot(x_ref[...], w1_ref[...], preferred_element_type=jnp.float32)
    y_ref[...] = xw * dinv[:N]


def _tc_b_body(accp_ref, y_ref, dinv_ref, b1_ref, gamma_ref, beta_ref,
               u_ref):
    dinv = dinv_ref[:N]
    s = dinv * (accp_ref[0, :N] + accp_ref[1, :N] + y_ref[...]) + b1_ref[...]
    mean = jnp.mean(s, axis=0, keepdims=True)
    var = jnp.mean((s - mean) * (s - mean), axis=0, keepdims=True)
    h = (s - mean) * (gamma_ref[...] / jnp.sqrt(var + 1e-5)) + beta_ref[...]
    h = jnp.maximum(h, 0.0)
    u_ref[...] = h * dinv


def _tc_c_body(accp_ref, u_ref, dinv_ref, w2_ref, b2_ref, out_ref):
    t = accp_ref[0, :N] + accp_ref[1, :N] + u_ref[...]
    out_ref[...] = (dinv_ref[:N]
                    * jnp.dot(t, w2_ref[...], preferred_element_type=jnp.float32)
                    + b2_ref[...])


def kernel(x, edge_index, W1, b1, gamma, beta, W2, b2):
    edge_flat = edge_index.reshape(2 * E)
    degp = _sc_degree(edge_flat)
    degp = degp.reshape(NC, NPAD, 1)

    y, dinv = pl.pallas_call(
        _tc_a_body,
        out_shape=(jax.ShapeDtypeStruct((N, D1), jnp.float32),
                   jax.ShapeDtypeStruct((NPAD, 1), jnp.float32)),
    )(x, W1, degp)

    zeros1 = jnp.zeros((RPT, D1), jnp.float32)
    accp1 = _sc_prop1(edge_flat, y, zeros1)

    u = pl.pallas_call(
        _tc_b_body,
        out_shape=jax.ShapeDtypeStruct((N, D1), jnp.float32),
    )(accp1, y, dinv, b1.reshape(1, D1), gamma.reshape(1, D1),
      beta.reshape(1, D1))

    accp2 = _sc_prop2(edge_flat, u, zeros1)

    out = pl.pallas_call(
        _tc_c_body,
        out_shape=jax.ShapeDtypeStruct((N, D2), jnp.float32),
    )(accp2, u, dinv, W2, b2.reshape(1, D2))
    return out
